# baseline (speedup 1.0000x reference)
import functools

import jax
import jax.numpy as jnp
from jax import lax
from jax.experimental import pallas as pl
from jax.experimental.pallas import tpu as pltpu
from jax.experimental.pallas import tpu_sc as plsc

_NC = 2
_NS = 16
_NW = _NC * _NS
_GATHER_CHUNK = 128


def _sc_gather(table, idx_flat):
    n = idx_flat.shape[0]
    d = table.shape[1]
    per_w = n // _NW
    chunks = per_w // _GATHER_CHUNK
    mesh = plsc.VectorSubcoreMesh(core_axis_name="c", subcore_axis_name="s")

    @functools.partial(
        pl.kernel,
        mesh=mesh,
        out_type=jax.ShapeDtypeStruct((n, d), jnp.float32),
        scratch_types=[
            pltpu.VMEM((per_w,), jnp.int32),
            pltpu.VMEM((_GATHER_CHUNK, d), jnp.float32),
            pltpu.SemaphoreType.DMA,
        ],
        compiler_params=pltpu.CompilerParams(use_tc_tiling_on_sc=False),
    )
    def gk(table_hbm, idx_hbm, out_hbm, idx_v, rows_v, sem):
        wid = lax.axis_index("s") * _NC + lax.axis_index("c")
        base = wid * per_w
        pltpu.sync_copy(idx_hbm.at[pl.ds(base, per_w)], idx_v)

        def body(j, carry):
            off = j * _GATHER_CHUNK
            pltpu.async_copy(
                table_hbm.at[idx_v.at[pl.ds(off, _GATHER_CHUNK)]], rows_v, sem
            ).wait()
            pltpu.sync_copy(rows_v, out_hbm.at[pl.ds(base + off, _GATHER_CHUNK)])
            return carry

        lax.fori_loop(0, chunks, body, 0)

    return gk(table, idx_flat)


def _tc_body(g_ref, i_ref, o_ref, wi_ref, bi_ref, wo_ref, bo_ref, w_ref,
             bout_ref, out_ref):
    t1 = jnp.dot(i_ref[...], wi_ref[...], preferred_element_type=jnp.float32)
    t1 = t1 + bi_ref[...]
    t2 = jnp.dot(o_ref[...], wo_ref[...], preferred_element_type=jnp.float32)
    t2 = t2 + bo_ref[...]
    acc = jnp.dot(g_ref[...], w_ref[0:64, :], preferred_element_type=jnp.float32)
    acc = acc + jnp.dot(t1, w_ref[64:96, :], preferred_element_type=jnp.float32)
    acc = acc + jnp.dot(t2, w_ref[96:128, :], preferred_element_type=jnp.float32)
    out_ref[...] = acc + bout_ref[...]


def _tc_dense(gath, ing2, oth2, w_ing, b_ing, w_o, b_o, w_out, b_out, tb=2048):
    n = gath.shape[0]
    grid = n // tb
    return pl.pallas_call(
        _tc_body,
        grid=(grid,),
        in_specs=[
            pl.BlockSpec((tb, 64), lambda i: (i, 0)),
            pl.BlockSpec((tb, 128), lambda i: (i, 0)),
            pl.BlockSpec((tb, 64), lambda i: (i, 0)),
            pl.BlockSpec((128, 32), lambda i: (0, 0)),
            pl.BlockSpec((1, 32), lambda i: (0, 0)),
            pl.BlockSpec((64, 32), lambda i: (0, 0)),
            pl.BlockSpec((1, 32), lambda i: (0, 0)),
            pl.BlockSpec((128, 128), lambda i: (0, 0)),
            pl.BlockSpec((1, 128), lambda i: (0, 0)),
        ],
        out_specs=pl.BlockSpec((tb, 128), lambda i: (i, 0)),
        out_shape=jax.ShapeDtypeStruct((n, 128), jnp.float32),
    )(gath, ing2, oth2, w_ing, b_ing.reshape(1, -1), w_o, b_o.reshape(1, -1),
      w_out, b_out.reshape(1, -1))


def kernel(recipe_id, ing, other_features, table, W_ing, b_ing, W_o, b_o,
           W_out, b_out):
    b, l = recipe_id.shape
    n = b * l
    idx = recipe_id.reshape(n)
    gath = _sc_gather(table, idx)
    ing2 = ing.reshape(n, ing.shape[-1])
    oth2 = other_features.reshape(n, other_features.shape[-1])
    out = _tc_dense(gath, ing2, oth2, W_ing, b_ing, W_o, b_o, W_out, b_out)
    return out.reshape(b, l, out.shape[-1])


# speedup vs baseline: 1.3655x; 1.3655x over previous
import functools

import jax
import jax.numpy as jnp
from jax import lax
from jax.experimental import pallas as pl
from jax.experimental.pallas import tpu as pltpu
from jax.experimental.pallas import tpu_sc as plsc

_NC = 2
_NS = 16
_NW = _NC * _NS
_CH = 640
_K = 16


def _sc_gather(table, idx_flat):
    n = idx_flat.shape[0]
    d = table.shape[1]
    per_w = n // _NW
    chunks = per_w // _CH
    mesh = plsc.VectorSubcoreMesh(core_axis_name="c", subcore_axis_name="s")

    @functools.partial(
        pl.kernel,
        mesh=mesh,
        out_type=jax.ShapeDtypeStruct((n, d), jnp.float32),
        scratch_types=[
            pltpu.VMEM((_CH,), jnp.int32),
            pltpu.VMEM((_CH, d), jnp.float32),
            pltpu.SemaphoreType.DMA,
            pltpu.SemaphoreType.DMA,
        ],
    )
    def gk(table_hbm, idx_hbm, out_hbm, idx_v, rows_v, gsem, isem):
        wid = lax.axis_index("s") * _NC + lax.axis_index("c")
        base = wid * per_w

        def chunk_body(c, carry):
            coff = pl.multiple_of(base + c * _CH, _CH)
            cpi = pltpu.make_async_copy(
                idx_hbm.at[pl.ds(coff, _CH)], idx_v, isem
            )
            cpi.start()
            cpi.wait()

            def fire_drain(b, carry2):
                boff = b * _K
                vec = idx_v[pl.ds(boff, _K)]
                copies = []
                for t in range(_K):
                    row = vec[t]
                    cp = pltpu.make_async_copy(
                        table_hbm.at[pl.ds(row, 1), :],
                        rows_v.at[pl.ds(boff + t, 1), :],
                        gsem,
                    )
                    cp.start()
                    copies.append(cp)
                for cp in copies:
                    cp.wait()
                return carry2

            lax.fori_loop(0, _CH // _K, fire_drain, 0)
            cpo = pltpu.make_async_copy(
                rows_v, out_hbm.at[pl.ds(coff, _CH)], isem
            )
            cpo.start()
            cpo.wait()
            return carry

        lax.fori_loop(0, chunks, chunk_body, 0)

    return gk(table, idx_flat)


def _tc_body(bb, g_ref, i_ref, o_ref, wi_ref, bi_ref, wo_ref, bo_ref, w_ref,
             bout_ref, out_ref):
    toks = bb * 50
    g = g_ref[...]
    i2 = i_ref[...].reshape(toks, 128)
    o2 = o_ref[...].reshape(toks, 64)
    t1 = jnp.dot(i2, wi_ref[...], preferred_element_type=jnp.float32)
    t1 = t1 + bi_ref[...]
    t2 = jnp.dot(o2, wo_ref[...], preferred_element_type=jnp.float32)
    t2 = t2 + bo_ref[...]
    acc = jnp.dot(g, w_ref[0:64, :], preferred_element_type=jnp.float32)
    acc = acc + jnp.dot(t1, w_ref[64:96, :], preferred_element_type=jnp.float32)
    acc = acc + jnp.dot(t2, w_ref[96:128, :], preferred_element_type=jnp.float32)
    acc = acc + bout_ref[...]
    out_ref[...] = acc.reshape(bb, 50, 128)


def _tc_dense(gath2, ing, oth, w_ing, b_ing, w_o, b_o, w_out, b_out, bb=64):
    b, l, _ = ing.shape
    grid = b // bb
    return pl.pallas_call(
        functools.partial(_tc_body, bb),
        grid=(grid,),
        in_specs=[
            pl.BlockSpec((bb * l, 64), lambda i: (i, 0)),
            pl.BlockSpec((bb, l, 128), lambda i: (i, 0, 0)),
            pl.BlockSpec((bb, l, 64), lambda i: (i, 0, 0)),
            pl.BlockSpec((128, 32), lambda i: (0, 0)),
            pl.BlockSpec((1, 32), lambda i: (0, 0)),
            pl.BlockSpec((64, 32), lambda i: (0, 0)),
            pl.BlockSpec((1, 32), lambda i: (0, 0)),
            pl.BlockSpec((128, 128), lambda i: (0, 0)),
            pl.BlockSpec((1, 128), lambda i: (0, 0)),
        ],
        out_specs=pl.BlockSpec((bb, l, 128), lambda i: (i, 0, 0)),
        out_shape=jax.ShapeDtypeStruct((b, l, 128), jnp.float32),
    )(gath2, ing, oth, w_ing, b_ing.reshape(1, -1), w_o, b_o.reshape(1, -1),
      w_out, b_out.reshape(1, -1))


def kernel(recipe_id, ing, other_features, table, W_ing, b_ing, W_o, b_o,
           W_out, b_out):
    b, l = recipe_id.shape
    n = b * l
    idx = recipe_id.reshape(n)
    gath2 = _sc_gather(table, idx)
    return _tc_dense(gath2, ing, other_features, W_ing, b_ing, W_o, b_o,
                     W_out, b_out)


# speedup vs baseline: 2.0871x; 1.5285x over previous
import functools

import jax
import jax.numpy as jnp
from jax import lax
from jax.experimental import pallas as pl
from jax.experimental.pallas import tpu as pltpu
from jax.experimental.pallas import tpu_sc as plsc

_NC = 2
_NS = 16
_NW = _NC * _NS
_G = 128


def _tr_body(in_ref, out_ref):
    y = jnp.swapaxes(in_ref[...], 0, 1)
    out_ref[...] = jnp.concatenate([y, y], axis=1)


def _tc_transpose(t64, tb=4096):
    v = t64.shape[1]
    grid = pl.cdiv(v, tb)
    return pl.pallas_call(
        _tr_body,
        grid=(grid,),
        in_specs=[pl.BlockSpec((64, tb), lambda i: (0, i))],
        out_specs=pl.BlockSpec((tb, 128), lambda i: (i, 0)),
        out_shape=jax.ShapeDtypeStruct((v, 128), jnp.float32),
    )(t64)


def _sc_gather(staged, idx_flat):
    n = idx_flat.shape[0]
    per_w = n // _NW
    chunks = per_w // _G
    mesh = plsc.VectorSubcoreMesh(core_axis_name="c", subcore_axis_name="s")

    @functools.partial(
        pl.kernel,
        mesh=mesh,
        out_type=jax.ShapeDtypeStruct((n, 128), jnp.float32),
        scratch_types=[
            pltpu.VMEM((per_w,), jnp.int32),
            pltpu.VMEM((_G, 128), jnp.float32),
            pltpu.SemaphoreType.DMA,
        ],
        compiler_params=pltpu.CompilerParams(use_tc_tiling_on_sc=False),
    )
    def gk(tab_hbm, idx_hbm, out_hbm, idx_v, rows_v, sem):
        wid = lax.axis_index("s") * _NC + lax.axis_index("c")
        base = wid * per_w
        pltpu.sync_copy(idx_hbm.at[pl.ds(base, per_w)], idx_v)

        def body(j, carry):
            off = j * _G
            pltpu.async_copy(
                tab_hbm.at[idx_v.at[pl.ds(off, _G)]], rows_v, sem
            ).wait()
            pltpu.sync_copy(rows_v, out_hbm.at[pl.ds(base + off, _G)])
            return carry

        lax.fori_loop(0, chunks, body, 0)

    return gk(staged, idx_flat)


def _tc_body(g_ref, i_ref, o_ref, wi_ref, bi_ref, wo_ref, bo_ref, w_ref,
             bout_ref, out_ref):
    t1 = jnp.dot(i_ref[...], wi_ref[...], preferred_element_type=jnp.float32)
    t1 = t1 + bi_ref[...]
    o2 = o_ref[0]
    t2 = lax.dot_general(o2, wo_ref[...], (((0,), (0,)), ((), ())),
                         preferred_element_type=jnp.float32)
    t2 = t2 + bo_ref[...]
    acc = jnp.dot(g_ref[:, 0:64], w_ref[0:64, :],
                  preferred_element_type=jnp.float32)
    acc = acc + jnp.dot(t1, w_ref[64:96, :], preferred_element_type=jnp.float32)
    acc = acc + jnp.dot(t2, w_ref[96:128, :], preferred_element_type=jnp.float32)
    out_ref[...] = acc + bout_ref[...]


def _tc_dense(gath, ing2, oth3, w_ing, b_ing, w_o, b_o, w_out, b_out):
    l, _, b = oth3.shape
    n = b * l
    return pl.pallas_call(
        _tc_body,
        grid=(l,),
        in_specs=[
            pl.BlockSpec((b, 128), lambda s: (s, 0)),
            pl.BlockSpec((b, 128), lambda s: (s, 0)),
            pl.BlockSpec((1, 64, b), lambda s: (s, 0, 0)),
            pl.BlockSpec((128, 32), lambda s: (0, 0)),
            pl.BlockSpec((1, 32), lambda s: (0, 0)),
            pl.BlockSpec((64, 32), lambda s: (0, 0)),
            pl.BlockSpec((1, 32), lambda s: (0, 0)),
            pl.BlockSpec((128, 128), lambda s: (0, 0)),
            pl.BlockSpec((1, 128), lambda s: (0, 0)),
        ],
        out_specs=pl.BlockSpec((b, 128), lambda s: (s, 0)),
        out_shape=jax.ShapeDtypeStruct((n, 128), jnp.float32),
        compiler_params=pltpu.CompilerParams(
            fuse_transposed_lhs_in_matmul=True),
    )(gath, ing2, oth3, w_ing, b_ing.reshape(1, -1), w_o, b_o.reshape(1, -1),
      w_out, b_out.reshape(1, -1))


def kernel(recipe_id, ing, other_features, table, W_ing, b_ing, W_o, b_o,
           W_out, b_out):
    b, l = recipe_id.shape
    n = b * l
    idx = recipe_id.T.reshape(n)
    staged = _tc_transpose(table.T)
    gath = _sc_gather(staged, idx)
    ing2 = ing.transpose(1, 0, 2).reshape(n, 128)
    oth3 = other_features.transpose(1, 2, 0)
    out2 = _tc_dense(gath, ing2, oth3, W_ing, b_ing, W_o, b_o, W_out, b_out)
    return out2.reshape(l, b, 128).transpose(1, 0, 2)


# speedup vs baseline: 2.2346x; 1.0707x over previous
import functools

import jax
import jax.numpy as jnp
from jax import lax
from jax.experimental import pallas as pl
from jax.experimental.pallas import tpu as pltpu
from jax.experimental.pallas import tpu_sc as plsc

_NC = 2
_NS = 16
_NW = _NC * _NS
_G = 128


def _tr_body(in_ref, out_ref):
    r = lax.broadcasted_iota(jnp.int32, (64, 128), 0)
    c = lax.broadcasted_iota(jnp.int32, (64, 128), 1)
    eye2 = (r == c % 64).astype(jnp.float32)
    out_ref[...] = lax.dot_general(
        in_ref[...], eye2, (((0,), (0,)), ((), ())),
        preferred_element_type=jnp.float32)


def _tc_transpose(t64, tb=4096):
    v = t64.shape[1]
    grid = pl.cdiv(v, tb)
    return pl.pallas_call(
        _tr_body,
        grid=(grid,),
        in_specs=[pl.BlockSpec((64, tb), lambda i: (0, i))],
        out_specs=pl.BlockSpec((tb, 128), lambda i: (i, 0)),
        out_shape=jax.ShapeDtypeStruct((v, 128), jnp.float32),
    )(t64)


def _sc_gather(staged, idx_flat):
    n = idx_flat.shape[0]
    per_w = n // _NW
    chunks = per_w // _G
    mesh = plsc.VectorSubcoreMesh(core_axis_name="c", subcore_axis_name="s")

    @functools.partial(
        pl.kernel,
        mesh=mesh,
        out_type=jax.ShapeDtypeStruct((n, 128), jnp.float32),
        scratch_types=[
            pltpu.VMEM((per_w,), jnp.int32),
            pltpu.VMEM((_G, 128), jnp.float32),
            pltpu.SemaphoreType.DMA,
        ],
        compiler_params=pltpu.CompilerParams(use_tc_tiling_on_sc=False),
    )
    def gk(tab_hbm, idx_hbm, out_hbm, idx_v, rows_v, sem):
        wid = lax.axis_index("s") * _NC + lax.axis_index("c")
        base = wid * per_w
        pltpu.sync_copy(idx_hbm.at[pl.ds(base, per_w)], idx_v)

        def body(j, carry):
            off = j * _G
            pltpu.async_copy(
                tab_hbm.at[idx_v.at[pl.ds(off, _G)]], rows_v, sem
            ).wait()
            pltpu.sync_copy(rows_v, out_hbm.at[pl.ds(base + off, _G)])
            return carry

        lax.fori_loop(0, chunks, body, 0)

    return gk(staged, idx_flat)


def _tc_body(g_ref, i_ref, o_ref, wi_ref, bi_ref, wo_ref, bo_ref, w_ref,
             bout_ref, out_ref):
    t1 = jnp.dot(i_ref[...], wi_ref[...], preferred_element_type=jnp.float32)
    t1 = t1 + bi_ref[...]
    o2 = o_ref[0]
    t2 = lax.dot_general(o2, wo_ref[...], (((0,), (0,)), ((), ())),
                         preferred_element_type=jnp.float32)
    t2 = t2 + bo_ref[...]
    acc = jnp.dot(g_ref[:, 0:64], w_ref[0:64, :],
                  preferred_element_type=jnp.float32)
    acc = acc + jnp.dot(t1, w_ref[64:96, :], preferred_element_type=jnp.float32)
    acc = acc + jnp.dot(t2, w_ref[96:128, :], preferred_element_type=jnp.float32)
    out_ref[...] = acc + bout_ref[...]


def _tc_dense(gath, ing2, oth3, w_ing, b_ing, w_o, b_o, w_out, b_out):
    l, _, b = oth3.shape
    n = b * l
    return pl.pallas_call(
        _tc_body,
        grid=(l,),
        in_specs=[
            pl.BlockSpec((b, 128), lambda s: (s, 0)),
            pl.BlockSpec((b, 128), lambda s: (s, 0)),
            pl.BlockSpec((1, 64, b), lambda s: (s, 0, 0)),
            pl.BlockSpec((128, 32), lambda s: (0, 0)),
            pl.BlockSpec((1, 32), lambda s: (0, 0)),
            pl.BlockSpec((64, 32), lambda s: (0, 0)),
            pl.BlockSpec((1, 32), lambda s: (0, 0)),
            pl.BlockSpec((128, 128), lambda s: (0, 0)),
            pl.BlockSpec((1, 128), lambda s: (0, 0)),
        ],
        out_specs=pl.BlockSpec((b, 128), lambda s: (s, 0)),
        out_shape=jax.ShapeDtypeStruct((n, 128), jnp.float32),
        compiler_params=pltpu.CompilerParams(
            fuse_transposed_lhs_in_matmul=True),
    )(gath, ing2, oth3, w_ing, b_ing.reshape(1, -1), w_o, b_o.reshape(1, -1),
      w_out, b_out.reshape(1, -1))


def kernel(recipe_id, ing, other_features, table, W_ing, b_ing, W_o, b_o,
           W_out, b_out):
    b, l = recipe_id.shape
    n = b * l
    idx = recipe_id.T.reshape(n)
    staged = _tc_transpose(table.T)
    gath = _sc_gather(staged, idx)
    ing2 = ing.transpose(1, 0, 2).reshape(n, 128)
    oth3 = other_features.transpose(1, 2, 0)
    out2 = _tc_dense(gath, ing2, oth3, W_ing, b_ing, W_o, b_o, W_out, b_out)
    return out2.reshape(l, b, 128).transpose(1, 0, 2)


# speedup vs baseline: 2.2780x; 1.0194x over previous
import functools

import jax
import jax.numpy as jnp
from jax import lax
from jax.experimental import pallas as pl
from jax.experimental.pallas import tpu as pltpu
from jax.experimental.pallas import tpu_sc as plsc

_NC = 2
_NS = 16
_NW = _NC * _NS
_G = 128


def _tr_body(in_ref, out_ref):
    r = lax.broadcasted_iota(jnp.int32, (64, 128), 0)
    c = lax.broadcasted_iota(jnp.int32, (64, 128), 1)
    eye2 = (r == c % 64).astype(jnp.float32)
    out_ref[...] = lax.dot_general(
        in_ref[...], eye2, (((0,), (0,)), ((), ())),
        preferred_element_type=jnp.float32)


def _tc_transpose(t64, tb=4096):
    v = t64.shape[1]
    grid = pl.cdiv(v, tb)
    return pl.pallas_call(
        _tr_body,
        grid=(grid,),
        in_specs=[pl.BlockSpec((64, tb), lambda i: (0, i))],
        out_specs=pl.BlockSpec((tb, 128), lambda i: (i, 0)),
        out_shape=jax.ShapeDtypeStruct((v, 128), jnp.float32),
    )(t64)


def _sc_gather(staged, idx_flat):
    n = idx_flat.shape[0]
    per_w = n // _NW
    chunks = per_w // _G
    mesh = plsc.VectorSubcoreMesh(core_axis_name="c", subcore_axis_name="s")

    @functools.partial(
        pl.kernel,
        mesh=mesh,
        out_type=jax.ShapeDtypeStruct((n, 128), jnp.float32),
        scratch_types=[
            pltpu.VMEM((per_w,), jnp.int32),
            pltpu.VMEM((_G, 128), jnp.float32),
            pltpu.SemaphoreType.DMA,
        ],
        compiler_params=pltpu.CompilerParams(use_tc_tiling_on_sc=False),
    )
    def gk(tab_hbm, idx_hbm, out_hbm, idx_v, rows_v, sem):
        wid = lax.axis_index("s") * _NC + lax.axis_index("c")
        base = wid * per_w
        pltpu.sync_copy(idx_hbm.at[pl.ds(base, per_w)], idx_v)

        def body(j, carry):
            off = j * _G
            pltpu.async_copy(
                tab_hbm.at[idx_v.at[pl.ds(off, _G)]], rows_v, sem
            ).wait()
            pltpu.sync_copy(rows_v, out_hbm.at[pl.ds(base + off, _G)])
            return carry

        lax.fori_loop(0, chunks, body, 0)

    return gk(staged, idx_flat)


def _tc_body(g_ref, i_ref, o_ref, wi_ref, bi_ref, wo_ref, bo_ref, w_ref,
             bout_ref, out_ref):
    t1 = jnp.dot(i_ref[...], wi_ref[...], preferred_element_type=jnp.float32)
    t1 = t1 + bi_ref[...]
    o2 = o_ref[0]
    t2 = lax.dot_general(o2, wo_ref[...], (((0,), (0,)), ((), ())),
                         preferred_element_type=jnp.float32)
    t2 = t2 + bo_ref[...]
    acc = jnp.dot(g_ref[:, 0:64], w_ref[0:64, :],
                  preferred_element_type=jnp.float32)
    acc = acc + jnp.dot(t1, w_ref[64:96, :], preferred_element_type=jnp.float32)
    acc = acc + jnp.dot(t2, w_ref[96:128, :], preferred_element_type=jnp.float32)
    out_ref[...] = acc + bout_ref[...]


def _tc_dense_chunk(prev, gath, ing2, oth3, w_ing, b_ing, w_o, b_o, w_out,
                    b_out, s_off, s_cnt):
    l, _, b = oth3.shape
    n = b * l

    def body(*refs):
        if prev is not None:
            refs = refs[1:]
        _tc_body(*refs)

    specs = [
        pl.BlockSpec((b, 128), lambda s: (s, 0)),
        pl.BlockSpec((b, 128), lambda s: (s_off + s, 0)),
        pl.BlockSpec((1, 64, b), lambda s: (s_off + s, 0, 0)),
        pl.BlockSpec((128, 32), lambda s: (0, 0)),
        pl.BlockSpec((1, 32), lambda s: (0, 0)),
        pl.BlockSpec((64, 32), lambda s: (0, 0)),
        pl.BlockSpec((1, 32), lambda s: (0, 0)),
        pl.BlockSpec((128, 128), lambda s: (0, 0)),
        pl.BlockSpec((1, 128), lambda s: (0, 0)),
    ]
    args = [gath, ing2, oth3, w_ing, b_ing.reshape(1, -1), w_o,
            b_o.reshape(1, -1), w_out, b_out.reshape(1, -1)]
    aliases = {}
    if prev is not None:
        specs = [pl.BlockSpec((b, 128), lambda s: (s_off + s, 0))] + specs
        args = [prev] + args
        aliases = {0: 0}

    return pl.pallas_call(
        body,
        grid=(s_cnt,),
        in_specs=specs,
        out_specs=pl.BlockSpec((b, 128), lambda s: (s_off + s, 0)),
        out_shape=jax.ShapeDtypeStruct((n, 128), jnp.float32),
        input_output_aliases=aliases,
        compiler_params=pltpu.CompilerParams(
            fuse_transposed_lhs_in_matmul=True),
    )(*args)


def kernel(recipe_id, ing, other_features, table, W_ing, b_ing, W_o, b_o,
           W_out, b_out):
    b, l = recipe_id.shape
    n = b * l
    idx = recipe_id.T.reshape(n)
    staged = _tc_transpose(table.T)
    ing2 = ing.transpose(1, 0, 2).reshape(n, 128)
    oth3 = other_features.transpose(1, 2, 0)
    nchunks = 2
    cs = l // nchunks
    out2 = None
    for k in range(nchunks):
        g = _sc_gather(staged, idx[k * cs * b:(k + 1) * cs * b])
        out2 = _tc_dense_chunk(out2, g, ing2, oth3, W_ing, b_ing, W_o, b_o,
                               W_out, b_out, k * cs, cs)
    return out2.reshape(l, b, 128).transpose(1, 0, 2)
